# baseline (speedup 1.0000x reference)
import jax
import jax.numpy as jnp
from jax import lax
from jax.experimental import pallas as pl
from jax.experimental.pallas import tpu as pltpu

_B = 16
_F = 7
_T = 4096
_CHUNK = 1024
_UNIFORM = 0.1429
_EPS = 1e-5


def _body(len_ref, t_ref, y_ref, out_ref, acc_ref, rowcap_ref):
    c = pl.program_id(0)

    @pl.when(c == 0)
    def _init():
        acc_ref[0] = 0.0
        rowcap_ref[...] = jnp.zeros((_B, 1), jnp.float32)

    t = t_ref[...]
    y = y_ref[...]
    lncol = len_ref[...].reshape(_B, 1)

    alleq = None
    psum = jnp.zeros((_B, _CHUNK), jnp.float32)
    for f in range(_F):
        tf = t[f]
        e = (tf >= 0.14285001) & (tf <= 0.14294998)
        alleq = e if f == 0 else (alleq & e)
        lg = jnp.where(tf > 0.0, jnp.log(tf), 0.0)
        psum = psum + tf * (lg - y[f])

    kv = lncol.astype(jnp.float32) * 0.0 + 1429.0
    k1 = (kv / 10000.0) == jnp.float32(_UNIFORM)

    tidx = lax.broadcasted_iota(jnp.int32, (_B, _CHUNK), 1) + c * _CHUNK
    valid = tidx < lncol
    cap = jnp.where((~(alleq & k1)) & valid, 1.0, 0.0)

    acc_ref[0] += jnp.sum(psum * cap)
    rowcap_ref[...] += jnp.sum(cap, axis=1, keepdims=True)

    @pl.when(c == pl.num_programs(0) - 1)
    def _fin():
        counter = jnp.sum(jnp.where(rowcap_ref[...] > 0.0, 1.0, 0.0))
        out_ref[0] = (jnp.float32(_EPS) + acc_ref[0]) / counter


def kernel(target, output, length):
    B, T, F = target.shape
    tt = jnp.transpose(target, (2, 0, 1))
    yt = jnp.transpose(output, (2, 0, 1))
    out = pl.pallas_call(
        _body,
        grid=(T // _CHUNK,),
        in_specs=[
            pl.BlockSpec((B,), lambda c: (0,)),
            pl.BlockSpec((F, B, _CHUNK), lambda c: (0, 0, c)),
            pl.BlockSpec((F, B, _CHUNK), lambda c: (0, 0, c)),
        ],
        out_specs=pl.BlockSpec(memory_space=pltpu.SMEM),
        out_shape=jax.ShapeDtypeStruct((1,), jnp.float32),
        scratch_shapes=[
            pltpu.SMEM((1,), jnp.float32),
            pltpu.VMEM((_B, 1), jnp.float32),
        ],
    )(length.astype(jnp.int32), tt, yt)
    return out[0]


# speedup vs baseline: 1.2083x; 1.2083x over previous
import jax
import jax.numpy as jnp
from jax import lax
from jax.experimental import pallas as pl
from jax.experimental.pallas import tpu as pltpu

_B = 16
_F = 7
_T = 4096
_CHUNK = 2048
_UNIFORM = 0.1429
_EPS = 1e-5


def _body(len_ref, t_ref, y_ref, out_ref, acc_ref, rowcap_ref):
    c = pl.program_id(0)

    @pl.when(c == 0)
    def _init():
        acc_ref[0] = 0.0
        rowcap_ref[...] = jnp.zeros((_B, 1), jnp.float32)

    t = t_ref[...]
    y = y_ref[...]
    lncol = len_ref[...].reshape(_B, 1)

    alleq = None
    psum = jnp.zeros((_B, _CHUNK), jnp.float32)
    for f in range(_F):
        tf = t[f]
        e = (tf >= 0.14285001) & (tf <= 0.14294998)
        alleq = e if f == 0 else (alleq & e)
        lg = jnp.where(tf > 0.0, jnp.log(tf), 0.0)
        psum = psum + tf * (lg - y[f])

    kv = lncol.astype(jnp.float32) * 0.0 + 1429.0
    k1 = (kv / 10000.0) == jnp.float32(_UNIFORM)

    tidx = lax.broadcasted_iota(jnp.int32, (_B, _CHUNK), 1) + c * _CHUNK
    valid = tidx < lncol
    cap = jnp.where((~(alleq & k1)) & valid, 1.0, 0.0)

    acc_ref[0] += jnp.sum(psum * cap)
    rowcap_ref[...] += jnp.sum(cap, axis=1, keepdims=True)

    @pl.when(c == pl.num_programs(0) - 1)
    def _fin():
        counter = jnp.sum(jnp.where(rowcap_ref[...] > 0.0, 1.0, 0.0))
        out_ref[0] = (jnp.float32(_EPS) + acc_ref[0]) / counter


def kernel(target, output, length):
    B, T, F = target.shape
    tt = jnp.transpose(target, (2, 0, 1))
    yt = jnp.transpose(output, (2, 0, 1))
    out = pl.pallas_call(
        _body,
        grid=(T // _CHUNK,),
        in_specs=[
            pl.BlockSpec((B,), lambda c: (0,)),
            pl.BlockSpec((F, B, _CHUNK), lambda c: (0, 0, c)),
            pl.BlockSpec((F, B, _CHUNK), lambda c: (0, 0, c)),
        ],
        out_specs=pl.BlockSpec(memory_space=pltpu.SMEM),
        out_shape=jax.ShapeDtypeStruct((1,), jnp.float32),
        scratch_shapes=[
            pltpu.SMEM((1,), jnp.float32),
            pltpu.VMEM((_B, 1), jnp.float32),
        ],
    )(length.astype(jnp.int32), tt, yt)
    return out[0]
